# baseline (speedup 1.0000x reference)
import functools

import jax
import jax.numpy as jnp
from jax import lax
from jax.experimental import pallas as pl
from jax.experimental.pallas import tpu as pltpu
from jax.experimental.pallas import tpu_sc as plsc

_N = 10000
_D = 128
_E = 320000
_CHUNK = 128
_NW = 32
_K = 80
_K0 = 160
_K1 = 0
_STAGE = 16
_E_PAD = _NW * _K * _CHUNK
_ROWS_PER_TILE = 632
_N_ACC = _ROWS_PER_TILE * 16
_ROW_BLK = 1000

_sc_mesh = plsc.VectorSubcoreMesh(core_axis_name="c", subcore_axis_name="s")


@functools.partial(
    pl.kernel, mesh=_sc_mesh,
    out_type=jax.ShapeDtypeStruct((2, _N_ACC, _D), jnp.float32),
    scratch_types=[
        pltpu.VMEM((_K, _CHUNK), jnp.int32),
        pltpu.VMEM((_CHUNK, _D), jnp.float32),
        pltpu.VMEM_SHARED((_N_ACC, _D), jnp.float32),
    ])
def _deg_pass(dst_hbm, d_hbm, dstv, ones, dacc):
    c = lax.axis_index("c")
    s = lax.axis_index("s")
    w = c * 16 + s
    base = s * _ROWS_PER_TILE
    pltpu.sync_copy(dst_hbm.at[pl.ds(w * _K, _K)], dstv)

    def _zf(i, carry):
        for t in range(_D // 16):
            ones[i, pl.ds(t * 16, 16)] = jnp.zeros((16,), jnp.float32)
        return carry
    lax.fori_loop(0, _CHUNK, _zf, 0)
    for off in range(0, _ROWS_PER_TILE, _CHUNK):
        sz = min(_CHUNK, _ROWS_PER_TILE - off)
        pltpu.sync_copy(ones.at[pl.ds(0, sz)],
                        dacc.at[pl.ds(base + off, sz)])

    def _of(i, carry):
        for t in range(_D // 16):
            ones[i, pl.ds(t * 16, 16)] = jnp.ones((16,), jnp.float32)
        return carry
    lax.fori_loop(0, _CHUNK, _of, 0)
    plsc.subcore_barrier()

    def _body(j, carry):
        pltpu.sync_copy(ones, dacc.at[dstv.at[j]], add=True)
        return carry
    lax.fori_loop(0, _K, _body, 0)

    plsc.subcore_barrier()
    pltpu.sync_copy(dacc.at[pl.ds(base, _ROWS_PER_TILE)],
                    d_hbm.at[c, pl.ds(base, _ROWS_PER_TILE)])


@functools.partial(
    pl.kernel, mesh=_sc_mesh,
    out_type=jax.ShapeDtypeStruct((2, _N_ACC, _D), jnp.float32),
    scratch_types=[
        pltpu.VMEM((_STAGE, _CHUNK), jnp.int32),
        pltpu.VMEM((_STAGE, _CHUNK), jnp.int32),
        pltpu.VMEM((_CHUNK, _D), jnp.float32),
        pltpu.VMEM((_CHUNK, _D), jnp.float32),
        pltpu.VMEM_SHARED((_N_ACC, _D), jnp.float32),
        pltpu.SemaphoreType.DMA,
        pltpu.SemaphoreType.DMA,
    ])
def _edge_pass(y_hbm, src_hbm, dst_hbm, p_hbm, srcv, dstv, bufa, bufb, acc,
               sem_a, sem_b):
    c = lax.axis_index("c")
    s = lax.axis_index("s")
    base = s * _ROWS_PER_TILE
    bufs = (bufa, bufb)
    sems = (sem_a, sem_b)
    start = jnp.where(c == 0, _K0 * s, 16 * _K0 + _K1 * s)
    n_stages = jnp.where(c == 0, _K0 // _STAGE, _K1 // _STAGE)

    def _zrow(i, carry):
        for t in range(_D // 16):
            bufa[i, pl.ds(t * 16, 16)] = jnp.zeros((16,), jnp.float32)
        return carry
    lax.fori_loop(0, _CHUNK, _zrow, 0)
    for off in range(0, _ROWS_PER_TILE, _CHUNK):
        sz = min(_CHUNK, _ROWS_PER_TILE - off)
        pltpu.sync_copy(bufa.at[pl.ds(0, sz)],
                        acc.at[pl.ds(base + off, sz)])
    plsc.subcore_barrier()

    def _gather(jj, buf, sem):
        pltpu.async_copy(y_hbm.at[srcv.at[jj]], buf, sem)

    def _gwait(jj, buf, sem):
        pltpu.make_async_copy(y_hbm.at[srcv.at[jj]], buf, sem).wait()

    @pl.when(n_stages > 0)
    def _():
        pltpu.sync_copy(src_hbm.at[pl.ds(start, _STAGE)], srcv)
        pltpu.sync_copy(dst_hbm.at[pl.ds(start, _STAGE)], dstv)
        _gather(0, bufa, sem_a)
        _gather(1, bufb, sem_b)

    def _stage(t, carry):
        def _body(i, carry):
            for b in range(2):
                jj = 2 * i + b
                _gwait(jj, bufs[b], sems[b])
                pltpu.sync_copy(bufs[b], acc.at[dstv.at[jj]], add=True)

                @pl.when(jj + 2 < _STAGE)
                def _():
                    _gather(jj + 2, bufs[b], sems[b])
            return carry
        lax.fori_loop(0, _STAGE // 2, _body, 0)

        @pl.when(t + 1 < n_stages)
        def _():
            off = start + (t + 1) * _STAGE
            pltpu.sync_copy(src_hbm.at[pl.ds(off, _STAGE)], srcv)
            pltpu.sync_copy(dst_hbm.at[pl.ds(off, _STAGE)], dstv)
            _gather(0, bufa, sem_a)
            _gather(1, bufb, sem_b)
        return carry
    lax.fori_loop(0, n_stages, _stage, 0)

    plsc.subcore_barrier()
    pltpu.sync_copy(acc.at[pl.ds(base, _ROWS_PER_TILE)],
                    p_hbm.at[c, pl.ds(base, _ROWS_PER_TILE)])


def _mm_body(x_ref, w_ref, o_ref):
    o_ref[...] = jnp.dot(x_ref[...], w_ref[...],
                         preferred_element_type=jnp.float32)


def _tc_matmul(x, w):
    return pl.pallas_call(
        _mm_body,
        grid=(_N // _ROW_BLK,),
        in_specs=[pl.BlockSpec((_ROW_BLK, _D), lambda i: (i, 0)),
                  pl.BlockSpec((_D, _D), lambda i: (0, 0))],
        out_specs=pl.BlockSpec((_ROW_BLK, _D), lambda i: (i, 0)),
        out_shape=jax.ShapeDtypeStruct((_N, _D), jnp.float32),
    )(x, w)


def _mid_body(p_ref, d_ref, b_ref, w_ref, o_ref):
    agg = p_ref[0] + p_ref[1]
    deg = d_ref[0] + d_ref[1]
    inv = 1.0 / jnp.maximum(deg[:, 0:1], 1.0)
    h = jnp.maximum(agg * inv + b_ref[...], 0.0)
    o_ref[...] = jnp.dot(h, w_ref[...], preferred_element_type=jnp.float32)


def _tc_mid(p, d, b, w):
    return pl.pallas_call(
        _mid_body,
        grid=(_N // _ROW_BLK,),
        in_specs=[pl.BlockSpec((2, _ROW_BLK, _D), lambda i: (0, i, 0)),
                  pl.BlockSpec((2, _ROW_BLK, _D), lambda i: (0, i, 0)),
                  pl.BlockSpec((1, _D), lambda i: (0, 0)),
                  pl.BlockSpec((_D, _D), lambda i: (0, 0))],
        out_specs=pl.BlockSpec((_ROW_BLK, _D), lambda i: (i, 0)),
        out_shape=jax.ShapeDtypeStruct((_N, _D), jnp.float32),
    )(p, d, b, w)


def _fin_body(q_ref, d_ref, b_ref, o_ref):
    agg = q_ref[0] + q_ref[1]
    deg = d_ref[0] + d_ref[1]
    o_ref[...] = agg / jnp.maximum(deg[:, 0:1], 1.0) + b_ref[...]


def _tc_fin(q, d, b):
    return pl.pallas_call(
        _fin_body,
        grid=(_N // _ROW_BLK,),
        in_specs=[pl.BlockSpec((2, _ROW_BLK, _D), lambda i: (0, i, 0)),
                  pl.BlockSpec((2, _ROW_BLK, _D), lambda i: (0, i, 0)),
                  pl.BlockSpec((1, _D), lambda i: (0, 0))],
        out_specs=pl.BlockSpec((_ROW_BLK, _D), lambda i: (i, 0)),
        out_shape=jax.ShapeDtypeStruct((_N, _D), jnp.float32),
    )(q, d, b)


def kernel(embedded_nodes, edges, W1, b1, W2, b2):
    pad = _E_PAD - _E
    srcp = jnp.concatenate(
        [edges[0], jnp.zeros((pad,), jnp.int32)]).reshape(_NW * _K, _CHUNK)
    dst_pad = _N + jnp.arange(pad, dtype=jnp.int32) % (_N_ACC - _N)
    dstp = jnp.concatenate([edges[1], dst_pad]).reshape(_NW * _K, _CHUNK)
    b1r = b1.reshape(1, _D)
    b2r = b2.reshape(1, _D)

    dsum = _deg_pass(dstp)
    y1 = _tc_matmul(embedded_nodes, W1)
    p = _edge_pass(y1, srcp, dstp)
    d_t = dsum[:, :_N, :]
    y2 = _tc_mid(p[:, :_N, :], d_t, b1r, W2)
    q = _edge_pass(y2, srcp, dstp)
    return _tc_fin(q[:, :_N, :], d_t, b2r)


# speedup vs baseline: 1.1033x; 1.1033x over previous
import functools

import jax
import jax.numpy as jnp
from jax import lax
from jax.experimental import pallas as pl
from jax.experimental.pallas import tpu as pltpu
from jax.experimental.pallas import tpu_sc as plsc

_N = 10000
_D = 128
_E = 320000
_CHUNK = 128
_NW = 32
_K = 80
_K0 = 80
_K1 = 80
_STAGE = 16
_E_PAD = _NW * _K * _CHUNK
_ROWS_PER_TILE = 632
_N_ACC = _ROWS_PER_TILE * 16
_ROW_BLK = 1000

_sc_mesh = plsc.VectorSubcoreMesh(core_axis_name="c", subcore_axis_name="s")


@functools.partial(
    pl.kernel, mesh=_sc_mesh,
    out_type=jax.ShapeDtypeStruct((2, _N_ACC, _D), jnp.float32),
    scratch_types=[
        pltpu.VMEM((_K, _CHUNK), jnp.int32),
        pltpu.VMEM((_CHUNK, _D), jnp.float32),
        pltpu.VMEM_SHARED((_N_ACC, _D), jnp.float32),
    ])
def _deg_pass(dst_hbm, d_hbm, dstv, ones, dacc):
    c = lax.axis_index("c")
    s = lax.axis_index("s")
    w = c * 16 + s
    base = s * _ROWS_PER_TILE
    pltpu.sync_copy(dst_hbm.at[pl.ds(w * _K, _K)], dstv)

    def _zf(i, carry):
        for t in range(_D // 16):
            ones[i, pl.ds(t * 16, 16)] = jnp.zeros((16,), jnp.float32)
        return carry
    lax.fori_loop(0, _CHUNK, _zf, 0)
    for off in range(0, _ROWS_PER_TILE, _CHUNK):
        sz = min(_CHUNK, _ROWS_PER_TILE - off)
        pltpu.sync_copy(ones.at[pl.ds(0, sz)],
                        dacc.at[pl.ds(base + off, sz)])

    def _of(i, carry):
        for t in range(_D // 16):
            ones[i, pl.ds(t * 16, 16)] = jnp.ones((16,), jnp.float32)
        return carry
    lax.fori_loop(0, _CHUNK, _of, 0)
    plsc.subcore_barrier()

    def _body(j, carry):
        pltpu.sync_copy(ones, dacc.at[dstv.at[j]], add=True)
        return carry
    lax.fori_loop(0, _K, _body, 0)

    plsc.subcore_barrier()
    pltpu.sync_copy(dacc.at[pl.ds(base, _ROWS_PER_TILE)],
                    d_hbm.at[c, pl.ds(base, _ROWS_PER_TILE)])


@functools.partial(
    pl.kernel, mesh=_sc_mesh,
    out_type=jax.ShapeDtypeStruct((2, _N_ACC, _D), jnp.float32),
    scratch_types=[
        pltpu.VMEM((_STAGE, _CHUNK), jnp.int32),
        pltpu.VMEM((_STAGE, _CHUNK), jnp.int32),
        pltpu.VMEM((_CHUNK, _D), jnp.float32),
        pltpu.VMEM((_CHUNK, _D), jnp.float32),
        pltpu.VMEM_SHARED((_N_ACC, _D), jnp.float32),
        pltpu.SemaphoreType.DMA,
        pltpu.SemaphoreType.DMA,
    ])
def _edge_pass(y_hbm, src_hbm, dst_hbm, p_hbm, srcv, dstv, bufa, bufb, acc,
               sem_a, sem_b):
    c = lax.axis_index("c")
    s = lax.axis_index("s")
    base = s * _ROWS_PER_TILE
    bufs = (bufa, bufb)
    sems = (sem_a, sem_b)
    start = jnp.where(c == 0, _K0 * s, 16 * _K0 + _K1 * s)
    n_stages = jnp.where(c == 0, _K0 // _STAGE, _K1 // _STAGE)

    def _zrow(i, carry):
        for t in range(_D // 16):
            bufa[i, pl.ds(t * 16, 16)] = jnp.zeros((16,), jnp.float32)
        return carry
    lax.fori_loop(0, _CHUNK, _zrow, 0)
    for off in range(0, _ROWS_PER_TILE, _CHUNK):
        sz = min(_CHUNK, _ROWS_PER_TILE - off)
        pltpu.sync_copy(bufa.at[pl.ds(0, sz)],
                        acc.at[pl.ds(base + off, sz)])
    plsc.subcore_barrier()

    def _gather(jj, buf, sem):
        pltpu.async_copy(y_hbm.at[c].at[srcv.at[jj]], buf, sem)

    def _gwait(jj, buf, sem):
        pltpu.make_async_copy(y_hbm.at[c].at[srcv.at[jj]], buf, sem).wait()

    @pl.when(n_stages > 0)
    def _():
        pltpu.sync_copy(src_hbm.at[pl.ds(start, _STAGE)], srcv)
        pltpu.sync_copy(dst_hbm.at[pl.ds(start, _STAGE)], dstv)
        _gather(0, bufa, sem_a)
        _gather(1, bufb, sem_b)

    def _stage(t, carry):
        def _body(i, carry):
            for b in range(2):
                jj = 2 * i + b
                _gwait(jj, bufs[b], sems[b])
                pltpu.sync_copy(bufs[b], acc.at[dstv.at[jj]], add=True)

                @pl.when(jj + 2 < _STAGE)
                def _():
                    _gather(jj + 2, bufs[b], sems[b])
            return carry
        lax.fori_loop(0, _STAGE // 2, _body, 0)

        @pl.when(t + 1 < n_stages)
        def _():
            off = start + (t + 1) * _STAGE
            pltpu.sync_copy(src_hbm.at[pl.ds(off, _STAGE)], srcv)
            pltpu.sync_copy(dst_hbm.at[pl.ds(off, _STAGE)], dstv)
            _gather(0, bufa, sem_a)
            _gather(1, bufb, sem_b)
        return carry
    lax.fori_loop(0, n_stages, _stage, 0)

    plsc.subcore_barrier()
    pltpu.sync_copy(acc.at[pl.ds(base, _ROWS_PER_TILE)],
                    p_hbm.at[c, pl.ds(base, _ROWS_PER_TILE)])


def _mm_body(x_ref, w_ref, o_ref):
    r = jnp.dot(x_ref[...], w_ref[...], preferred_element_type=jnp.float32)
    o_ref[0] = r
    o_ref[1] = r


def _tc_matmul(x, w):
    return pl.pallas_call(
        _mm_body,
        grid=(_N // _ROW_BLK,),
        in_specs=[pl.BlockSpec((_ROW_BLK, _D), lambda i: (i, 0)),
                  pl.BlockSpec((_D, _D), lambda i: (0, 0))],
        out_specs=pl.BlockSpec((2, _ROW_BLK, _D), lambda i: (0, i, 0)),
        out_shape=jax.ShapeDtypeStruct((2, _N, _D), jnp.float32),
    )(x, w)


def _mid_body(p_ref, d_ref, b_ref, w_ref, o_ref):
    agg = p_ref[0] + p_ref[1]
    deg = d_ref[0] + d_ref[1]
    inv = 1.0 / jnp.maximum(deg[:, 0:1], 1.0)
    h = jnp.maximum(agg * inv + b_ref[...], 0.0)
    r = jnp.dot(h, w_ref[...], preferred_element_type=jnp.float32)
    o_ref[0] = r
    o_ref[1] = r


def _tc_mid(p, d, b, w):
    return pl.pallas_call(
        _mid_body,
        grid=(_N // _ROW_BLK,),
        in_specs=[pl.BlockSpec((2, _ROW_BLK, _D), lambda i: (0, i, 0)),
                  pl.BlockSpec((2, _ROW_BLK, _D), lambda i: (0, i, 0)),
                  pl.BlockSpec((1, _D), lambda i: (0, 0)),
                  pl.BlockSpec((_D, _D), lambda i: (0, 0))],
        out_specs=pl.BlockSpec((2, _ROW_BLK, _D), lambda i: (0, i, 0)),
        out_shape=jax.ShapeDtypeStruct((2, _N, _D), jnp.float32),
    )(p, d, b, w)


def _fin_body(q_ref, d_ref, b_ref, o_ref):
    agg = q_ref[0] + q_ref[1]
    deg = d_ref[0] + d_ref[1]
    o_ref[...] = agg / jnp.maximum(deg[:, 0:1], 1.0) + b_ref[...]


def _tc_fin(q, d, b):
    return pl.pallas_call(
        _fin_body,
        grid=(_N // _ROW_BLK,),
        in_specs=[pl.BlockSpec((2, _ROW_BLK, _D), lambda i: (0, i, 0)),
                  pl.BlockSpec((2, _ROW_BLK, _D), lambda i: (0, i, 0)),
                  pl.BlockSpec((1, _D), lambda i: (0, 0))],
        out_specs=pl.BlockSpec((_ROW_BLK, _D), lambda i: (i, 0)),
        out_shape=jax.ShapeDtypeStruct((_N, _D), jnp.float32),
    )(q, d, b)


def kernel(embedded_nodes, edges, W1, b1, W2, b2):
    pad = _E_PAD - _E
    srcp = jnp.concatenate(
        [edges[0], jnp.zeros((pad,), jnp.int32)]).reshape(_NW * _K, _CHUNK)
    dst_pad = _N + jnp.arange(pad, dtype=jnp.int32) % (_N_ACC - _N)
    dstp = jnp.concatenate([edges[1], dst_pad]).reshape(_NW * _K, _CHUNK)
    b1r = b1.reshape(1, _D)
    b2r = b2.reshape(1, _D)

    dsum = _deg_pass(dstp)
    y1 = _tc_matmul(embedded_nodes, W1)
    p = _edge_pass(y1, srcp, dstp)
    d_t = dsum[:, :_N, :]
    y2 = _tc_mid(p[:, :_N, :], d_t, b1r, W2)
    q = _edge_pass(y2, srcp, dstp)
    return _tc_fin(q[:, :_N, :], d_t, b2r)


# speedup vs baseline: 1.3520x; 1.2254x over previous
import functools

import jax
import jax.numpy as jnp
from jax import lax
from jax.experimental import pallas as pl
from jax.experimental.pallas import tpu as pltpu
from jax.experimental.pallas import tpu_sc as plsc

_N = 10000
_D = 128
_E = 320000
_CHUNK = 128
_NW = 32
_K = 80
_K0 = 152
_K1 = 8
_STAGE = 8
_E_PAD = _NW * _K * _CHUNK
_ROWS_PER_TILE = 632
_N_ACC = _ROWS_PER_TILE * 16
_ROW_BLK = 1000

_sc_mesh = plsc.VectorSubcoreMesh(core_axis_name="c", subcore_axis_name="s")


@functools.partial(
    pl.kernel, mesh=_sc_mesh,
    out_type=jax.ShapeDtypeStruct((2, _N_ACC, _D), jnp.float32),
    scratch_types=[
        pltpu.VMEM((_K, _CHUNK), jnp.int32),
        pltpu.VMEM((_CHUNK, _D), jnp.float32),
        pltpu.VMEM_SHARED((_N_ACC, _D), jnp.float32),
    ])
def _deg_pass(dst_hbm, d_hbm, dstv, ones, dacc):
    c = lax.axis_index("c")
    s = lax.axis_index("s")
    w = c * 16 + s
    base = s * _ROWS_PER_TILE
    pltpu.sync_copy(dst_hbm.at[pl.ds(w * _K, _K)], dstv)

    def _zf(i, carry):
        for t in range(_D // 16):
            ones[i, pl.ds(t * 16, 16)] = jnp.zeros((16,), jnp.float32)
        return carry
    lax.fori_loop(0, _CHUNK, _zf, 0)
    for off in range(0, _ROWS_PER_TILE, _CHUNK):
        sz = min(_CHUNK, _ROWS_PER_TILE - off)
        pltpu.sync_copy(ones.at[pl.ds(0, sz)],
                        dacc.at[pl.ds(base + off, sz)])

    def _of(i, carry):
        for t in range(_D // 16):
            ones[i, pl.ds(t * 16, 16)] = jnp.ones((16,), jnp.float32)
        return carry
    lax.fori_loop(0, _CHUNK, _of, 0)
    plsc.subcore_barrier()

    def _body(j, carry):
        pltpu.sync_copy(ones, dacc.at[dstv.at[j]], add=True)
        return carry
    lax.fori_loop(0, _K, _body, 0)

    plsc.subcore_barrier()
    pltpu.sync_copy(dacc.at[pl.ds(base, _ROWS_PER_TILE)],
                    d_hbm.at[c, pl.ds(base, _ROWS_PER_TILE)])


@functools.partial(
    pl.kernel, mesh=_sc_mesh,
    out_type=jax.ShapeDtypeStruct((2, _N_ACC, _D), jnp.float32),
    scratch_types=[
        pltpu.VMEM((_STAGE, _CHUNK), jnp.int32),
        pltpu.VMEM((_STAGE, _CHUNK), jnp.int32),
        pltpu.VMEM((_CHUNK, _D), jnp.float32),
        pltpu.VMEM((_CHUNK, _D), jnp.float32),
        pltpu.VMEM_SHARED((_N_ACC, _D), jnp.float32),
        pltpu.SemaphoreType.DMA,
        pltpu.SemaphoreType.DMA,
    ])
def _edge_pass(y_hbm, src_hbm, dst_hbm, p_hbm, srcv, dstv, bufa, bufb, acc,
               sem_a, sem_b):
    c = lax.axis_index("c")
    s = lax.axis_index("s")
    base = s * _ROWS_PER_TILE
    bufs = (bufa, bufb)
    sems = (sem_a, sem_b)
    start = jnp.where(c == 0, _K0 * s, 16 * _K0 + _K1 * s)
    n_stages = jnp.where(c == 0, _K0 // _STAGE, _K1 // _STAGE)

    def _zrow(i, carry):
        for t in range(_D // 16):
            bufa[i, pl.ds(t * 16, 16)] = jnp.zeros((16,), jnp.float32)
        return carry
    lax.fori_loop(0, _CHUNK, _zrow, 0)
    for off in range(0, _ROWS_PER_TILE, _CHUNK):
        sz = min(_CHUNK, _ROWS_PER_TILE - off)
        pltpu.sync_copy(bufa.at[pl.ds(0, sz)],
                        acc.at[pl.ds(base + off, sz)])
    plsc.subcore_barrier()

    def _gather(jj, buf, sem):
        pltpu.async_copy(y_hbm.at[srcv.at[jj]], buf, sem)

    def _gwait(jj, buf, sem):
        pltpu.make_async_copy(y_hbm.at[srcv.at[jj]], buf, sem).wait()

    @pl.when(n_stages > 0)
    def _():
        pltpu.sync_copy(src_hbm.at[pl.ds(start, _STAGE)], srcv)
        pltpu.sync_copy(dst_hbm.at[pl.ds(start, _STAGE)], dstv)
        _gather(0, bufa, sem_a)
        _gather(1, bufb, sem_b)

    def _stage(t, carry):
        def _body(i, carry):
            for b in range(2):
                jj = 2 * i + b
                _gwait(jj, bufs[b], sems[b])
                pltpu.sync_copy(bufs[b], acc.at[dstv.at[jj]], add=True)

                @pl.when(jj + 2 < _STAGE)
                def _():
                    _gather(jj + 2, bufs[b], sems[b])
            return carry
        lax.fori_loop(0, _STAGE // 2, _body, 0)

        @pl.when(t + 1 < n_stages)
        def _():
            off = start + (t + 1) * _STAGE
            pltpu.sync_copy(src_hbm.at[pl.ds(off, _STAGE)], srcv)
            pltpu.sync_copy(dst_hbm.at[pl.ds(off, _STAGE)], dstv)
            _gather(0, bufa, sem_a)
            _gather(1, bufb, sem_b)
        return carry
    lax.fori_loop(0, n_stages, _stage, 0)

    plsc.subcore_barrier()
    pltpu.sync_copy(acc.at[pl.ds(base, _ROWS_PER_TILE)],
                    p_hbm.at[c, pl.ds(base, _ROWS_PER_TILE)])


def _mm_body(x_ref, w_ref, o_ref):
    o_ref[...] = jnp.dot(x_ref[...], w_ref[...],
                         preferred_element_type=jnp.float32)


def _tc_matmul(x, w):
    return pl.pallas_call(
        _mm_body,
        grid=(_N // _ROW_BLK,),
        in_specs=[pl.BlockSpec((_ROW_BLK, _D), lambda i: (i, 0)),
                  pl.BlockSpec((_D, _D), lambda i: (0, 0))],
        out_specs=pl.BlockSpec((_ROW_BLK, _D), lambda i: (i, 0)),
        out_shape=jax.ShapeDtypeStruct((_N, _D), jnp.float32),
    )(x, w)


def _mid_body(p_ref, d_ref, b_ref, w_ref, o_ref):
    agg = p_ref[0] + p_ref[1]
    deg = d_ref[0] + d_ref[1]
    inv = 1.0 / jnp.maximum(deg[:, 0:1], 1.0)
    h = jnp.maximum(agg * inv + b_ref[...], 0.0)
    o_ref[...] = jnp.dot(h, w_ref[...], preferred_element_type=jnp.float32)


def _tc_mid(p, d, b, w):
    return pl.pallas_call(
        _mid_body,
        grid=(_N // _ROW_BLK,),
        in_specs=[pl.BlockSpec((2, _ROW_BLK, _D), lambda i: (0, i, 0)),
                  pl.BlockSpec((2, _ROW_BLK, _D), lambda i: (0, i, 0)),
                  pl.BlockSpec((1, _D), lambda i: (0, 0)),
                  pl.BlockSpec((_D, _D), lambda i: (0, 0))],
        out_specs=pl.BlockSpec((_ROW_BLK, _D), lambda i: (i, 0)),
        out_shape=jax.ShapeDtypeStruct((_N, _D), jnp.float32),
    )(p, d, b, w)


def _fin_body(q_ref, d_ref, b_ref, o_ref):
    agg = q_ref[0] + q_ref[1]
    deg = d_ref[0] + d_ref[1]
    o_ref[...] = agg / jnp.maximum(deg[:, 0:1], 1.0) + b_ref[...]


def _tc_fin(q, d, b):
    return pl.pallas_call(
        _fin_body,
        grid=(_N // _ROW_BLK,),
        in_specs=[pl.BlockSpec((2, _ROW_BLK, _D), lambda i: (0, i, 0)),
                  pl.BlockSpec((2, _ROW_BLK, _D), lambda i: (0, i, 0)),
                  pl.BlockSpec((1, _D), lambda i: (0, 0))],
        out_specs=pl.BlockSpec((_ROW_BLK, _D), lambda i: (i, 0)),
        out_shape=jax.ShapeDtypeStruct((_N, _D), jnp.float32),
    )(q, d, b)


def kernel(embedded_nodes, edges, W1, b1, W2, b2):
    pad = _E_PAD - _E
    srcp = jnp.concatenate(
        [edges[0], jnp.zeros((pad,), jnp.int32)]).reshape(_NW * _K, _CHUNK)
    dst_pad = _N + jnp.arange(pad, dtype=jnp.int32) % (_N_ACC - _N)
    dstp = jnp.concatenate([edges[1], dst_pad]).reshape(_NW * _K, _CHUNK)
    b1r = b1.reshape(1, _D)
    b2r = b2.reshape(1, _D)

    dsum = _deg_pass(dstp)
    y1 = _tc_matmul(embedded_nodes, W1)
    p = _edge_pass(y1, srcp, dstp)
    d_t = dsum[:, :_N, :]
    y2 = _tc_mid(p[:, :_N, :], d_t, b1r, W2)
    q = _edge_pass(y2, srcp, dstp)
    return _tc_fin(q[:, :_N, :], d_t, b2r)
